# baseline (speedup 1.0000x reference)
import functools

import jax
import jax.numpy as jnp
from jax import lax
from jax.experimental import pallas as pl
from jax.experimental.pallas import tpu as pltpu, tpu_sc as plsc

B = 16384
K = 32
NC = 2
NS = 16
NW = NC * NS
BPW = B // NW


def _body(xu_hbm, xi_hbm, wt_hbm, ht_hbm, out_hbm,
          idx_u, idx_i, u_kbuf, h_kbuf, out_v, sem):
  wid = lax.axis_index("s") * NC + lax.axis_index("c")
  base = wid * BPW

  pltpu.sync_copy(xu_hbm.at[pl.ds(base, BPW)], idx_u)
  pltpu.sync_copy(xi_hbm.at[pl.ds(base, BPW)], idx_i)

  copies = []
  for k in range(K):
    copies.append(
        pltpu.async_copy(wt_hbm.at[k].at[idx_u], u_kbuf.at[k], sem))
    copies.append(
        pltpu.async_copy(ht_hbm.at[k].at[idx_i], h_kbuf.at[k], sem))
  for c in copies:
    c.wait()

  iota = lax.iota(jnp.int32, 16)

  def group(c, _):
    col = pl.ds(c * 16, 16)
    acc = jnp.zeros((16,), jnp.float32)
    for k in range(K):
      acc = acc + u_kbuf[k, col] * h_kbuf[k, col]
    plsc.store_scatter(out_v, [c * 16 + iota], acc)
    return _

  lax.fori_loop(0, BPW // 16, group, None)

  pltpu.sync_copy(out_v, out_hbm.at[pl.ds(base, BPW)])


def kernel(x_user, x_item, W, H):
  xu = x_user.astype(jnp.int32)
  xi = x_item.astype(jnp.int32)
  wt = jnp.swapaxes(W, 0, 1)
  ht = jnp.swapaxes(H, 0, 1)

  mesh = plsc.VectorSubcoreMesh(core_axis_name="c", subcore_axis_name="s")
  k = functools.partial(
      pl.kernel,
      out_type=jax.ShapeDtypeStruct((B,), jnp.float32),
      mesh=mesh,
      compiler_params=pltpu.CompilerParams(
          needs_layout_passes=False, use_tc_tiling_on_sc=False),
      scratch_types=[
          pltpu.VMEM((BPW,), jnp.int32),
          pltpu.VMEM((BPW,), jnp.int32),
          pltpu.VMEM((K, BPW), jnp.float32),
          pltpu.VMEM((K, BPW), jnp.float32),
          pltpu.VMEM((BPW,), jnp.float32),
          pltpu.SemaphoreType.DMA,
      ],
  )(_body)
  return k(xu, xi, wt, ht)
